# baseline (speedup 1.0000x reference)
import functools

import jax
import jax.numpy as jnp
from jax import lax
from jax.experimental import pallas as pl
from jax.experimental.pallas import tpu as pltpu
from jax.experimental.pallas import tpu_sc as plsc

N, C, H, W = 16, 256, 56, 56
HW = H * W
HG, WG = 112, 112
G = HG * WG
HALF = G // 2
NBLK = HALF // 16
NROWCH = HG // 2 // 8
RCBLK = NBLK // NROWCH
VPR = WG // 16
CVEC = C // 16


def _body(feats, grid_h, out_h, gridv, idxb, w00, w01, w10, w11,
          rows_v, outst, gsem0, gsem1, osem0, osem1):
    wid = lax.axis_index("s") * 2 + lax.axis_index("c")
    n = wid // 2
    half = wid % 2
    gsem = (gsem0, gsem1)
    osem = (osem0, osem1)

    lanes = lax.iota(jnp.int32, 16)
    rowbase = n * HW
    orowbase = n * G + half * HALF

    def phase1_chunk(sub, carry):
        gy0 = half * (HG // 2) + sub * 8
        pltpu.sync_copy(grid_h.at[n, pl.ds(gy0, 8)], gridv)

        @plsc.parallel_loop(sub * RCBLK, (sub + 1) * RCBLK)
        def _phase1(blk):
            loc = blk - sub * RCBLK
            r = loc // VPR
            cc = (loc % VPR) * 16
            x = gridv[r, 0, pl.ds(cc, 16)]
            y = gridv[r, 1, pl.ds(cc, 16)]
            ix = ((x + 1.0) * W - 1.0) * 0.5
            iy = ((y + 1.0) * H - 1.0) * 0.5
            x0 = ix.astype(jnp.int32)
            x0 = jnp.where(x0.astype(jnp.float32) > ix, x0 - 1, x0)
            y0 = iy.astype(jnp.int32)
            y0 = jnp.where(y0.astype(jnp.float32) > iy, y0 - 1, y0)
            fx = ix - x0.astype(jnp.float32)
            fy = iy - y0.astype(jnp.float32)
            x1 = x0 + 1
            y1 = y0 + 1
            wx0 = jnp.where((x0 >= 0) & (x0 <= W - 1), 1.0 - fx, 0.0)
            wx1 = jnp.where((x1 >= 0) & (x1 <= W - 1), fx, 0.0)
            wy0 = jnp.where((y0 >= 0) & (y0 <= H - 1), 1.0 - fy, 0.0)
            wy1 = jnp.where((y1 >= 0) & (y1 <= H - 1), fy, 0.0)
            cx0 = jnp.clip(x0, 0, W - 1)
            cx1 = jnp.clip(x1, 0, W - 1)
            cy0 = jnp.clip(y0, 0, H - 1)
            cy1 = jnp.clip(y1, 0, H - 1)
            s = blk * 16
            pos = 4 * (s + lanes)
            plsc.store_scatter(idxb, [pos], rowbase + cy0 * W + cx0)
            plsc.store_scatter(idxb, [pos + 1], rowbase + cy1 * W + cx0)
            plsc.store_scatter(idxb, [pos + 2], rowbase + cy0 * W + cx1)
            plsc.store_scatter(idxb, [pos + 3], rowbase + cy1 * W + cx1)
            w00[pl.ds(s, 16)] = wy0 * wx0
            w01[pl.ds(s, 16)] = wy1 * wx0
            w10[pl.ds(s, 16)] = wy0 * wx1
            w11[pl.ds(s, 16)] = wy1 * wx1

        return carry

    lax.fori_loop(0, NROWCH, phase1_chunk, 0)

    def gather_copy(ch, b):
        return pltpu.make_async_copy(
            feats.at[idxb.at[pl.ds(ch * 64, 64)]],
            rows_v.at[pl.ds(b * 64, 64)], gsem[b])

    def out_copy(ch, b):
        return pltpu.make_async_copy(
            outst.at[pl.ds(b * 16, 16)],
            out_h.at[pl.ds(orowbase + ch * 16, 16)], osem[b])

    gather_copy(0, 0).start()

    def chunk_pair(i, carry):
        for b in (0, 1):
            ch = 2 * i + b

            @pl.when(ch < NBLK - 1)
            def _():
                gather_copy(ch + 1, 1 - b).start()

            gather_copy(ch, b).wait()

            @pl.when(i > 0)
            def _():
                out_copy(ch, b).wait()

            s = ch * 16
            wv0 = w00[pl.ds(s, 16)]
            wv1 = w01[pl.ds(s, 16)]
            wv2 = w10[pl.ds(s, 16)]
            wv3 = w11[pl.ds(s, 16)]
            for p in range(16):
                r0 = b * 64 + 4 * p
                ws0 = wv0[p]
                ws1 = wv1[p]
                ws2 = wv2[p]
                ws3 = wv3[p]
                for cc in range(CVEC):
                    acc = (rows_v[r0, pl.ds(cc * 16, 16)] * ws0
                           + rows_v[r0 + 1, pl.ds(cc * 16, 16)] * ws1
                           + rows_v[r0 + 2, pl.ds(cc * 16, 16)] * ws2
                           + rows_v[r0 + 3, pl.ds(cc * 16, 16)] * ws3)
                    outst[b * 16 + p, pl.ds(cc * 16, 16)] = acc

            out_copy(ch, b).start()
        return carry

    lax.fori_loop(0, NBLK // 2, chunk_pair, 0)
    out_copy(NBLK - 2, 0).wait()
    out_copy(NBLK - 1, 1).wait()


_sampler = functools.partial(
    pl.kernel,
    out_type=jax.ShapeDtypeStruct((N * G, C), jnp.float32),
    mesh=plsc.VectorSubcoreMesh(core_axis_name="c", subcore_axis_name="s"),
    compiler_params=pltpu.CompilerParams(needs_layout_passes=False),
    scratch_types=[
        pltpu.VMEM((8, 2, WG), jnp.float32),
        pltpu.VMEM((4 * HALF,), jnp.int32),
        pltpu.VMEM((HALF,), jnp.float32),
        pltpu.VMEM((HALF,), jnp.float32),
        pltpu.VMEM((HALF,), jnp.float32),
        pltpu.VMEM((HALF,), jnp.float32),
        pltpu.VMEM((2 * 64, C), jnp.float32),
        pltpu.VMEM((2 * 16, C), jnp.float32),
        pltpu.SemaphoreType.DMA,
        pltpu.SemaphoreType.DMA,
        pltpu.SemaphoreType.DMA,
        pltpu.SemaphoreType.DMA,
    ],
)(_body)


def kernel(input_features, grid):
    table = input_features.transpose(0, 2, 3, 1).reshape(N * HW, C)
    gt = grid.transpose(0, 1, 3, 2)
    out = _sampler(table, gt)
    return out.reshape(N, HG, WG, C).transpose(0, 3, 1, 2)


# speedup vs baseline: 1.1571x; 1.1571x over previous
import functools

import jax
import jax.numpy as jnp
from jax import lax
from jax.experimental import pallas as pl
from jax.experimental.pallas import tpu as pltpu
from jax.experimental.pallas import tpu_sc as plsc

N, C, H, W = 16, 256, 56, 56
NB = N // 2
HG, WG = 112, 112
G = HG * WG
HALF = G // 2
NBLK = HALF // 16
KPR = 4
KCH = 2 * KPR
NGRP = (C // 4) // KPR
NSUB = 7
SUBBLK = NBLK // NSUB
SUBPTS = SUBBLK * 16
SUBROWS = HG // 2 // NSUB
VPR = WG // 16
HI_MASK = jnp.int32(-65536)


def _body(feats, grid_h, out_h, gridv, xp, yp, w00, w01, w10, w11,
          imgv, outv, isem0, isem1, osem0, osem1):
    wid = lax.axis_index("s") * 2 + lax.axis_index("c")
    n = wid // 4
    half = (wid // 2) % 2
    chb = (wid % 2) * (C // 4)
    isem = (isem0, isem1)
    osem = (osem0, osem1)

    def phase1_chunk(sub, carry):
        gy0 = half * (HG // 2) + sub * SUBROWS
        pltpu.sync_copy(grid_h.at[n, pl.ds(gy0, SUBROWS)], gridv)

        @plsc.parallel_loop(sub * SUBBLK, (sub + 1) * SUBBLK)
        def _phase1(blk):
            loc = blk - sub * SUBBLK
            r = loc // VPR
            cc = (loc % VPR) * 16
            x = gridv[r, 0, pl.ds(cc, 16)]
            y = gridv[r, 1, pl.ds(cc, 16)]
            ix = ((x + 1.0) * W - 1.0) * 0.5
            iy = ((y + 1.0) * H - 1.0) * 0.5
            x0 = ix.astype(jnp.int32)
            x0 = jnp.where(x0.astype(jnp.float32) > ix, x0 - 1, x0)
            y0 = iy.astype(jnp.int32)
            y0 = jnp.where(y0.astype(jnp.float32) > iy, y0 - 1, y0)
            fx = ix - x0.astype(jnp.float32)
            fy = iy - y0.astype(jnp.float32)
            x1 = x0 + 1
            y1 = y0 + 1
            wx0 = jnp.where((x0 >= 0) & (x0 <= W - 1), 1.0 - fx, 0.0)
            wx1 = jnp.where((x1 >= 0) & (x1 <= W - 1), fx, 0.0)
            wy0 = jnp.where((y0 >= 0) & (y0 <= H - 1), 1.0 - fy, 0.0)
            wy1 = jnp.where((y1 >= 0) & (y1 <= H - 1), fy, 0.0)
            cx0 = jnp.clip(x0, 0, W - 1)
            cx1 = jnp.clip(x1, 0, W - 1)
            cy0 = jnp.clip(y0, 0, H - 1)
            cy1 = jnp.clip(y1, 0, H - 1)
            s = blk * 16
            xp[pl.ds(s, 16)] = cx0 | (cx1 << 16)
            yp[pl.ds(s, 16)] = cy0 | (cy1 << 16)
            w00[pl.ds(s, 16)] = wy0 * wx0
            w01[pl.ds(s, 16)] = wy1 * wx0
            w10[pl.ds(s, 16)] = wy0 * wx1
            w11[pl.ds(s, 16)] = wy1 * wx1

        return carry

    lax.fori_loop(0, NSUB, phase1_chunk, 0)

    def img_copy(g, b):
        return pltpu.make_async_copy(
            feats.at[n, pl.ds(chb + g * KPR, KPR)],
            imgv.at[pl.ds(b * KPR, KPR)], isem[b])

    img_copy(0, 0).start()

    def group_pair(i, carry):
        for b in (0, 1):
            g = 2 * i + b

            @pl.when(g < NGRP - 1)
            def _():
                img_copy(g + 1, 1 - b).start()

            img_copy(g, b).wait()

            for sub in range(NSUB):
                ob = (b + sub) % 2
                r0 = half * (HG // 2) + sub * SUBROWS

                def drain():
                    for k in range(KCH):
                        pltpu.make_async_copy(
                            outv.at[ob * KCH + k],
                            out_h.at[n, k, pl.ds(r0, SUBROWS)],
                            osem[ob]).wait()

                if sub < 2:
                    @pl.when(g > 0)
                    def _():
                        drain()
                else:
                    drain()

                @plsc.parallel_loop(sub * SUBBLK, (sub + 1) * SUBBLK)
                def _blkloop(blk):
                    s = blk * 16
                    px = xp[pl.ds(s, 16)]
                    py = yp[pl.ds(s, 16)]
                    ax0 = px & 0xFFFF
                    ax1 = lax.shift_right_logical(px, 16)
                    ay0 = py & 0xFFFF
                    ay1 = lax.shift_right_logical(py, 16)
                    b00 = w00[pl.ds(s, 16)]
                    b01 = w01[pl.ds(s, 16)]
                    b10 = w10[pl.ds(s, 16)]
                    b11 = w11[pl.ds(s, 16)]
                    blkloc = blk - sub * SUBBLK
                    r = blkloc // VPR
                    c0 = (blkloc % VPR) * 16

                    def expand(v):
                        lo = plsc.bitcast(v << 16, jnp.float32)
                        hi = plsc.bitcast(v & HI_MASK, jnp.float32)
                        return lo, hi

                    for k in range(KPR):
                        kv = jnp.full((16,), b * KPR + k, jnp.int32)
                        v00 = plsc.load_gather(imgv, [kv, ay0, ax0])
                        v01 = plsc.load_gather(imgv, [kv, ay1, ax0])
                        v10 = plsc.load_gather(imgv, [kv, ay0, ax1])
                        v11 = plsc.load_gather(imgv, [kv, ay1, ax1])
                        lo00, hi00 = expand(v00)
                        lo01, hi01 = expand(v01)
                        lo10, hi10 = expand(v10)
                        lo11, hi11 = expand(v11)
                        acc0 = (lo00 * b00 + lo01 * b01
                                + lo10 * b10 + lo11 * b11)
                        acc1 = (hi00 * b00 + hi01 * b01
                                + hi10 * b10 + hi11 * b11)
                        outv[ob * KCH + 2 * k, r, pl.ds(c0, 16)] = acc0
                        outv[ob * KCH + 2 * k + 1, r, pl.ds(c0, 16)] = acc1

                obase = 2 * chb + g * KCH
                for k in range(KCH):
                    pltpu.make_async_copy(
                        outv.at[ob * KCH + k],
                        out_h.at[n, obase + k, pl.ds(r0, SUBROWS)],
                        osem[ob]).start()
        return carry

    lax.fori_loop(0, NGRP // 2, group_pair, 0)

    def drain_final(ob):
        for k in range(KCH):
            pltpu.make_async_copy(
                outv.at[ob * KCH + k],
                out_h.at[n, k, pl.ds(0, SUBROWS)],
                osem[ob]).wait()

    drain_final(0)
    drain_final(1)


_sampler = functools.partial(
    pl.kernel,
    out_type=jax.ShapeDtypeStruct((NB, C, HG, WG), jnp.float32),
    mesh=plsc.VectorSubcoreMesh(core_axis_name="c", subcore_axis_name="s"),
    compiler_params=pltpu.CompilerParams(needs_layout_passes=False),
    scratch_types=[
        pltpu.VMEM((SUBROWS, 2, WG), jnp.float32),
        pltpu.VMEM((HALF,), jnp.int32),
        pltpu.VMEM((HALF,), jnp.int32),
        pltpu.VMEM((HALF,), jnp.float32),
        pltpu.VMEM((HALF,), jnp.float32),
        pltpu.VMEM((HALF,), jnp.float32),
        pltpu.VMEM((HALF,), jnp.float32),
        pltpu.VMEM((2 * KPR, H, W), jnp.int32),
        pltpu.VMEM((2 * KCH, SUBROWS, WG), jnp.float32),
        pltpu.SemaphoreType.DMA,
        pltpu.SemaphoreType.DMA,
        pltpu.SemaphoreType.DMA,
        pltpu.SemaphoreType.DMA,
    ],
)(_body)


def kernel(input_features, grid):
    outs = []
    for n0 in (0, NB):
        fb = input_features[n0:n0 + NB].astype(jnp.bfloat16)
        fb = fb.reshape(NB, C // 2, 2, H, W)
        fb = jnp.moveaxis(fb, 2, 4)
        packed = lax.bitcast_convert_type(fb, jnp.int32)
        gt = grid[n0:n0 + NB].transpose(0, 1, 3, 2)
        outs.append(_sampler(packed, gt))
    return jnp.concatenate(outs, axis=0)


# speedup vs baseline: 1.3091x; 1.1314x over previous
import functools

import jax
import jax.numpy as jnp
from jax import lax
from jax.experimental import pallas as pl
from jax.experimental.pallas import tpu as pltpu
from jax.experimental.pallas import tpu_sc as plsc

N, C, H, W = 16, 256, 56, 56
HG, WG = 112, 112
G = HG * WG
HALF = G // 2
NBLK = HALF // 16
KPR = 4
KCH = 2 * KPR
NGRP = (C // 2) // KPR
NSUB = 7
SUBBLK = NBLK // NSUB
SUBPTS = SUBBLK * 16
SUBROWS = HG // 2 // NSUB
VPR = WG // 16
HI_MASK = jnp.int32(-65536)


def _body(feats, grid_h, out_h, gridv, xp, yp, w00, w01, w10, w11,
          imgv, outv, isem0, isem1, osem0, osem1):
    wid = lax.axis_index("s") * 2 + lax.axis_index("c")
    n = wid // 2
    half = wid % 2
    isem = (isem0, isem1)
    osem = (osem0, osem1)

    def phase1_chunk(sub, carry):
        gy0 = half * (HG // 2) + sub * SUBROWS
        pltpu.sync_copy(grid_h.at[n, pl.ds(gy0, SUBROWS)], gridv)

        @plsc.parallel_loop(sub * SUBBLK, (sub + 1) * SUBBLK)
        def _phase1(blk):
            loc = blk - sub * SUBBLK
            r = loc // VPR
            cc = (loc % VPR) * 16
            x = gridv[r, 0, pl.ds(cc, 16)]
            y = gridv[r, 1, pl.ds(cc, 16)]
            ix = ((x + 1.0) * W - 1.0) * 0.5
            iy = ((y + 1.0) * H - 1.0) * 0.5
            x0 = ix.astype(jnp.int32)
            x0 = jnp.where(x0.astype(jnp.float32) > ix, x0 - 1, x0)
            y0 = iy.astype(jnp.int32)
            y0 = jnp.where(y0.astype(jnp.float32) > iy, y0 - 1, y0)
            fx = ix - x0.astype(jnp.float32)
            fy = iy - y0.astype(jnp.float32)
            x1 = x0 + 1
            y1 = y0 + 1
            wx0 = jnp.where((x0 >= 0) & (x0 <= W - 1), 1.0 - fx, 0.0)
            wx1 = jnp.where((x1 >= 0) & (x1 <= W - 1), fx, 0.0)
            wy0 = jnp.where((y0 >= 0) & (y0 <= H - 1), 1.0 - fy, 0.0)
            wy1 = jnp.where((y1 >= 0) & (y1 <= H - 1), fy, 0.0)
            cx0 = jnp.clip(x0, 0, W - 1)
            cx1 = jnp.clip(x1, 0, W - 1)
            cy0 = jnp.clip(y0, 0, H - 1)
            cy1 = jnp.clip(y1, 0, H - 1)
            s = blk * 16
            xp[pl.ds(s, 16)] = cx0 | (cx1 << 16)
            yp[pl.ds(s, 16)] = cy0 | (cy1 << 16)
            w00[pl.ds(s, 16)] = wy0 * wx0
            w01[pl.ds(s, 16)] = wy1 * wx0
            w10[pl.ds(s, 16)] = wy0 * wx1
            w11[pl.ds(s, 16)] = wy1 * wx1

        return carry

    lax.fori_loop(0, NSUB, phase1_chunk, 0)

    def img_copy(g, b):
        return pltpu.make_async_copy(
            feats.at[n, pl.ds(g * KPR, KPR)],
            imgv.at[pl.ds(b * KPR, KPR)], isem[b])

    img_copy(0, 0).start()

    def group_pair(i, carry):
        for b in (0, 1):
            g = 2 * i + b

            @pl.when(g < NGRP - 1)
            def _():
                img_copy(g + 1, 1 - b).start()

            img_copy(g, b).wait()

            for sub in range(NSUB):
                ob = (b + sub) % 2
                r0 = half * (HG // 2) + sub * SUBROWS

                def drain():
                    for k in range(KCH):
                        pltpu.make_async_copy(
                            outv.at[ob * KCH + k],
                            out_h.at[n, k, pl.ds(r0, SUBROWS)],
                            osem[ob]).wait()

                if sub < 2:
                    @pl.when(g > 0)
                    def _():
                        drain()
                else:
                    drain()

                @plsc.parallel_loop(sub * SUBBLK, (sub + 1) * SUBBLK)
                def _blkloop(blk):
                    s = blk * 16
                    px = xp[pl.ds(s, 16)]
                    py = yp[pl.ds(s, 16)]
                    ax0 = px & 0xFFFF
                    ax1 = lax.shift_right_logical(px, 16)
                    ay0 = py & 0xFFFF
                    ay1 = lax.shift_right_logical(py, 16)
                    b00 = w00[pl.ds(s, 16)]
                    b01 = w01[pl.ds(s, 16)]
                    b10 = w10[pl.ds(s, 16)]
                    b11 = w11[pl.ds(s, 16)]
                    blkloc = blk - sub * SUBBLK
                    r = blkloc // VPR
                    c0 = (blkloc % VPR) * 16

                    def expand(v):
                        lo = plsc.bitcast(v << 16, jnp.float32)
                        hi = plsc.bitcast(v & HI_MASK, jnp.float32)
                        return lo, hi

                    for k in range(KPR):
                        kv = jnp.full((16,), b * KPR + k, jnp.int32)
                        v00 = plsc.load_gather(imgv, [kv, ay0, ax0])
                        v01 = plsc.load_gather(imgv, [kv, ay1, ax0])
                        v10 = plsc.load_gather(imgv, [kv, ay0, ax1])
                        v11 = plsc.load_gather(imgv, [kv, ay1, ax1])
                        lo00, hi00 = expand(v00)
                        lo01, hi01 = expand(v01)
                        lo10, hi10 = expand(v10)
                        lo11, hi11 = expand(v11)
                        acc0 = (lo00 * b00 + lo01 * b01
                                + lo10 * b10 + lo11 * b11)
                        acc1 = (hi00 * b00 + hi01 * b01
                                + hi10 * b10 + hi11 * b11)
                        outv[ob * KCH + 2 * k, r, pl.ds(c0, 16)] = acc0
                        outv[ob * KCH + 2 * k + 1, r, pl.ds(c0, 16)] = acc1

                obase = g * KCH
                for k in range(KCH):
                    pltpu.make_async_copy(
                        outv.at[ob * KCH + k],
                        out_h.at[n, obase + k, pl.ds(r0, SUBROWS)],
                        osem[ob]).start()
        return carry

    lax.fori_loop(0, NGRP // 2, group_pair, 0)

    def drain_final(ob):
        for k in range(KCH):
            pltpu.make_async_copy(
                outv.at[ob * KCH + k],
                out_h.at[n, k, pl.ds(0, SUBROWS)],
                osem[ob]).wait()

    drain_final(0)
    drain_final(1)


_sampler = functools.partial(
    pl.kernel,
    out_type=jax.ShapeDtypeStruct((N, C, HG, WG), jnp.float32),
    mesh=plsc.VectorSubcoreMesh(core_axis_name="c", subcore_axis_name="s"),
    compiler_params=pltpu.CompilerParams(needs_layout_passes=False),
    scratch_types=[
        pltpu.VMEM((SUBROWS, 2, WG), jnp.float32),
        pltpu.VMEM((HALF,), jnp.int32),
        pltpu.VMEM((HALF,), jnp.int32),
        pltpu.VMEM((HALF,), jnp.float32),
        pltpu.VMEM((HALF,), jnp.float32),
        pltpu.VMEM((HALF,), jnp.float32),
        pltpu.VMEM((HALF,), jnp.float32),
        pltpu.VMEM((2 * KPR, H, W), jnp.int32),
        pltpu.VMEM((2 * KCH, SUBROWS, WG), jnp.float32),
        pltpu.SemaphoreType.DMA,
        pltpu.SemaphoreType.DMA,
        pltpu.SemaphoreType.DMA,
        pltpu.SemaphoreType.DMA,
    ],
)(_body)


def kernel(input_features, grid):
    u = lax.bitcast_convert_type(input_features, jnp.uint32)

    def rnd(x):
        return (x + 0x7FFF + ((x >> 16) & 1)) >> 16

    packed = lax.bitcast_convert_type(
        rnd(u[:, 0::2]) | (rnd(u[:, 1::2]) << 16), jnp.int32)
    gt = grid.transpose(0, 1, 3, 2)
    return _sampler(packed, gt)
